# Initial kernel scaffold; baseline (speedup 1.0000x reference)
import jax
import jax.numpy as jnp
from jax.experimental import pallas as pl


def kernel(node_features, edge_indices, edges_features, batch_size, emb_w, emb_b, e1_w, e1_b, e2_w, e2_b, n1_w, n1_b, n2_w, n2_b, u1_w, u1_b, u2_w, u2_b):
    raise NotImplementedError("write your pallas kernel here")


# speedup vs baseline: 2.0224x; 2.0224x over previous
import functools

import jax
import jax.numpy as jnp
from jax import lax
from jax.experimental import pallas as pl
from jax.experimental.pallas import tpu as pltpu
from jax.experimental.pallas import tpu_sc as plsc

NC = 2
NS = 16
NW = NC * NS
IB = 128
W = 128


def _mesh():
    return plsc.VectorSubcoreMesh(
        core_axis_name="c", subcore_axis_name="s", num_cores=NC,
        num_subcores=NS)


def _make_gather(n_pad, e_pad, cpw, gb):
    @functools.partial(
        pl.kernel,
        out_type=[jax.ShapeDtypeStruct((e_pad, W), jnp.float32),
                  jax.ShapeDtypeStruct((e_pad, W), jnp.float32)],
        mesh=_mesh(),
        scratch_types=[
            pltpu.VMEM((gb, IB), jnp.int32),
            pltpu.VMEM((gb, IB), jnp.int32),
            pltpu.VMEM((gb * IB, W), jnp.float32),
            pltpu.VMEM((gb * IB, W), jnp.float32),
            pltpu.SemaphoreType.DMA,
            pltpu.SemaphoreType.DMA,
        ],
    )
    def gather_k(x_hbm, srcb_hbm, dstb_hbm, xs_hbm, xd_hbm,
                 sidx, didx, srows, drows, sem_s, sem_d):
        cid = lax.axis_index("c")
        sid = lax.axis_index("s")
        wid = sid * NC + cid
        base_blk = wid * cpw

        def body(g, carry):
            blk0 = base_blk + g * gb
            pltpu.sync_copy(srcb_hbm.at[pl.ds(blk0, gb)], sidx)
            pltpu.sync_copy(dstb_hbm.at[pl.ds(blk0, gb)], didx)
            cps = [pltpu.async_copy(x_hbm.at[sidx.at[b]],
                                    srows.at[pl.ds(b * IB, IB)], sem_s)
                   for b in range(gb)]
            cpd = [pltpu.async_copy(x_hbm.at[didx.at[b]],
                                    drows.at[pl.ds(b * IB, IB)], sem_d)
                   for b in range(gb)]
            for c in cps:
                c.wait()
            for c in cpd:
                c.wait()
            pltpu.sync_copy(srows, xs_hbm.at[pl.ds(blk0 * IB, gb * IB)])
            pltpu.sync_copy(drows, xd_hbm.at[pl.ds(blk0 * IB, gb * IB)])
            return carry

        lax.fori_loop(0, cpw // gb, body, 0)

    return gather_k


def _make_scatter(n_pad, e_pad, cpw, gb):
    half = n_pad // 2
    acc_h = half + NS * 8
    rpt = half // NS
    zpt = acc_h // NS

    @functools.partial(
        pl.kernel,
        out_type=jax.ShapeDtypeStruct((n_pad, W), jnp.float32),
        mesh=_mesh(),
        scratch_types=[
            pltpu.VMEM((gb, IB), jnp.int32),
            pltpu.VMEM((gb * IB, W), jnp.float32),
            pltpu.VMEM_SHARED((acc_h, W), jnp.float32),
            pltpu.SemaphoreType.DMA,
        ],
    )
    def scatter_k(m_hbm, dstb_hbm, zeros_hbm, out_hbm,
                  didx, mrows, acc_sp, sem):
        cid = lax.axis_index("c")
        sid = lax.axis_index("s")
        wid = sid * NC + cid
        pltpu.sync_copy(zeros_hbm.at[pl.ds(sid * zpt, zpt)],
                        acc_sp.at[pl.ds(sid * zpt, zpt)])
        plsc.subcore_barrier()

        def body(g, carry):
            def run(eblk0):
                pltpu.sync_copy(dstb_hbm.at[cid, pl.ds(eblk0, gb)], didx)
                pltpu.async_copy(m_hbm.at[pl.ds(eblk0 * IB, gb * IB)],
                                 mrows, sem).wait()
                for b in range(gb):
                    pltpu.sync_copy(mrows.at[pl.ds(b * IB, IB)],
                                    acc_sp.at[didx.at[b]], add=True)

            run(wid * cpw + g * gb)
            run((wid ^ 1) * cpw + g * gb)
            return carry

        lax.fori_loop(0, cpw // gb, body, 0)
        plsc.subcore_barrier()
        pltpu.sync_copy(acc_sp.at[pl.ds(sid * rpt, rpt)],
                        out_hbm.at[pl.ds(cid * half + sid * rpt, rpt)])

    return scatter_k


def _embed_body(nf_ref, w_ref, b_ref, o_ref):
    x = jnp.dot(nf_ref[...], w_ref[...],
                preferred_element_type=jnp.float32) + b_ref[...]
    o_ref[...] = jnp.concatenate([x, jnp.zeros_like(x)], axis=1)


def _edge_body(xs_ref, xd_ref, ef_ref, ws_ref, wd_ref, we_ref, b1_ref,
               w2_ref, b2_ref, m_ref):
    h = xs_ref[...].shape[1] // 2
    xs = xs_ref[...][:, :h]
    xd = xd_ref[...][:, :h]
    z = (jnp.dot(xs, ws_ref[...], preferred_element_type=jnp.float32)
         + jnp.dot(xd, wd_ref[...], preferred_element_type=jnp.float32)
         + jnp.dot(ef_ref[...], we_ref[...], preferred_element_type=jnp.float32)
         + b1_ref[...])
    z = z * jax.nn.sigmoid(z)
    m = jnp.dot(z, w2_ref[...], preferred_element_type=jnp.float32) + b2_ref[...]
    m = m * jax.nn.sigmoid(m)
    m_ref[...] = jnp.concatenate([m, jnp.zeros_like(m)], axis=1)


def _node_body(x_ref, a0_ref, w1x_ref, w1a_ref, b1_ref, w2_ref,
               b2_ref, o_ref):
    h = x_ref[...].shape[1] // 2
    x = x_ref[...][:, :h]
    agg = a0_ref[...][:, :h]
    z = (jnp.dot(x, w1x_ref[...], preferred_element_type=jnp.float32)
         + jnp.dot(agg, w1a_ref[...], preferred_element_type=jnp.float32)
         + b1_ref[...])
    z = jnp.maximum(z, 0.0)
    xo = x + jnp.dot(z, w2_ref[...], preferred_element_type=jnp.float32) \
        + b2_ref[...]
    o_ref[...] = jnp.concatenate([xo, jnp.zeros_like(xo)], axis=1)


def _unembed_body(x_ref, w1_ref, b1_ref, w2_ref, b2_ref, o_ref):
    h = x_ref[...].shape[1] // 2
    x = x_ref[...][:, :h]
    z = jnp.dot(x, w1_ref[...],
                preferred_element_type=jnp.float32) + b1_ref[...]
    z = z * jax.nn.sigmoid(z)
    o_ref[...] = jnp.dot(z, w2_ref[...],
                         preferred_element_type=jnp.float32) + b2_ref[...]


def _row_blocked(body, n_rows, blk, in_shapes, out_cols):
    grid = (n_rows // blk,)
    in_specs = []
    for s in in_shapes:
        if s[0] == n_rows:
            in_specs.append(pl.BlockSpec((blk, s[1]), lambda i: (i, 0)))
        else:
            in_specs.append(pl.BlockSpec(s, lambda i, _r=len(s): (0,) * _r))
    return pl.pallas_call(
        body,
        grid=grid,
        in_specs=in_specs,
        out_specs=pl.BlockSpec((blk, out_cols), lambda i: (i, 0)),
        out_shape=jax.ShapeDtypeStruct((n_rows, out_cols), jnp.float32),
    )


def kernel(node_features, edge_indices, edges_features, batch_size,
           emb_w, emb_b, e1_w, e1_b, e2_w, e2_b,
           n1_w, n1_b, n2_w, n2_b, u1_w, u1_b, u2_w, u2_b):
    n, df = node_features.shape
    e = edge_indices.shape[1]
    de = edges_features.shape[1]
    h = emb_w.shape[0]
    n_layers = e1_w.shape[0]
    do = u2_w.shape[0]

    n_pad = ((n + 16 + 1023) // 1024) * 1024
    cpw = -(-e // (NW * IB))
    if cpw % 2:
        cpw += 1
    gb = 2
    e_pad = NW * cpw * IB
    bn = 1024
    be = 1024

    f32 = jnp.float32
    src = edge_indices[0]
    dst = edge_indices[1]
    pad_e = e_pad - e
    pad_ids = jnp.arange(pad_e, dtype=jnp.int32)
    src_p = jnp.concatenate([src, pad_ids % n]).reshape(e_pad // IB, IB)
    dst_f = jnp.concatenate([dst, n + (pad_ids % 16)])
    dst_p = dst_f.reshape(e_pad // IB, IB)
    half = n_pad // 2
    eidx = jnp.arange(e_pad, dtype=jnp.int32)
    dst_lo = jnp.where(dst_f < half, dst_f, half + (eidx % 16))
    dst_hi = jnp.where(dst_f >= half, dst_f - half, (n - half) + (eidx % 16))
    dst2 = jnp.stack([dst_lo, dst_hi]).reshape(2, e_pad // IB, IB)
    ef_p = jnp.zeros((e_pad, de), f32).at[:e].set(edges_features)
    nf_p = jnp.zeros((n_pad, df), f32).at[:n].set(node_features)
    zeros_np = jnp.zeros((half + NS * 8, W), f32)

    gather = _make_gather(n_pad, e_pad, cpw, gb)
    scatter = _make_scatter(n_pad, e_pad, cpw, gb)

    embed = _row_blocked(_embed_body, n_pad, bn,
                         [(n_pad, df), (df, h), (1, h)], W)
    edge_mlp = _row_blocked(_edge_body, e_pad, be,
                            [(e_pad, W), (e_pad, W), (e_pad, de),
                             (h, h), (h, h), (de, h), (1, h), (h, h), (1, h)],
                            W)
    node_mlp = _row_blocked(_node_body, n_pad, bn,
                            [(n_pad, W), (n_pad, W),
                             (h, h), (h, h), (1, h), (h, h), (1, h)], W)
    unembed = _row_blocked(_unembed_body, n_pad, bn,
                           [(n_pad, W), (h, h), (1, h), (h, do), (1, do)], do)

    x = embed(nf_p, emb_w.T, emb_b[None])
    for l in range(n_layers):
        ws = e1_w[l, :, :h].T
        wd = e1_w[l, :, h:2 * h].T
        we = e1_w[l, :, 2 * h:].T
        xs, xd = gather(x, src_p, dst_p)
        m = edge_mlp(xs, xd, ef_p, ws, wd, we, e1_b[l][None],
                     e2_w[l].T, e2_b[l][None])
        agg = scatter(m, dst2, zeros_np)
        x = node_mlp(x, agg, n1_w[l, :, :h].T, n1_w[l, :, h:].T,
                     n1_b[l][None], n2_w[l].T, n2_b[l][None])
    pred = unembed(x, u1_w.T, u1_b[None], u2_w.T, u2_b[None])[:n]
    return (pred, edge_indices, edges_features)


# speedup vs baseline: 2.3318x; 1.1530x over previous
import functools

import jax
import jax.numpy as jnp
from jax import lax
from jax.experimental import pallas as pl
from jax.experimental.pallas import tpu as pltpu
from jax.experimental.pallas import tpu_sc as plsc

NC = 2
NS = 16
NW = NC * NS
IB = 128
W = 128


def _mesh():
    return plsc.VectorSubcoreMesh(
        core_axis_name="c", subcore_axis_name="s", num_cores=NC,
        num_subcores=NS)


def _make_gather(n_pad, e_pad, cpw, gb):
    @functools.partial(
        pl.kernel,
        out_type=[jax.ShapeDtypeStruct((e_pad, W), jnp.float32),
                  jax.ShapeDtypeStruct((e_pad, W), jnp.float32)],
        mesh=_mesh(),
        scratch_types=[
            pltpu.VMEM((gb, IB), jnp.int32),
            pltpu.VMEM((gb, IB), jnp.int32),
            pltpu.VMEM((gb * IB, W), jnp.float32),
            pltpu.VMEM((gb * IB, W), jnp.float32),
            pltpu.SemaphoreType.DMA,
            pltpu.SemaphoreType.DMA,
        ],
    )
    def gather_k(x_hbm, srcb_hbm, dstb_hbm, xs_hbm, xd_hbm,
                 sidx, didx, srows, drows, sem_s, sem_d):
        cid = lax.axis_index("c")
        sid = lax.axis_index("s")
        wid = sid * NC + cid
        base_blk = wid * cpw

        def body(g, carry):
            blk0 = base_blk + g * gb
            pltpu.sync_copy(srcb_hbm.at[pl.ds(blk0, gb)], sidx)
            pltpu.sync_copy(dstb_hbm.at[pl.ds(blk0, gb)], didx)
            cps = [pltpu.async_copy(x_hbm.at[sidx.at[b]],
                                    srows.at[pl.ds(b * IB, IB)], sem_s)
                   for b in range(gb)]
            cpd = [pltpu.async_copy(x_hbm.at[didx.at[b]],
                                    drows.at[pl.ds(b * IB, IB)], sem_d)
                   for b in range(gb)]
            for c in cps:
                c.wait()
            for c in cpd:
                c.wait()
            pltpu.sync_copy(srows, xs_hbm.at[pl.ds(blk0 * IB, gb * IB)])
            pltpu.sync_copy(drows, xd_hbm.at[pl.ds(blk0 * IB, gb * IB)])
            return carry

        lax.fori_loop(0, cpw // gb, body, 0)

    return gather_k


def _make_scatter(n_pad, e_pad, cpw, gb):
    rpt = n_pad // NS

    @functools.partial(
        pl.kernel,
        out_type=jax.ShapeDtypeStruct((NC, n_pad, W), jnp.float32),
        mesh=_mesh(),
        scratch_types=[
            pltpu.VMEM((gb, IB), jnp.int32),
            pltpu.VMEM((gb * IB, W), jnp.float32),
            pltpu.VMEM_SHARED((n_pad, W), jnp.float32),
            pltpu.SemaphoreType.DMA,
        ],
    )
    def scatter_k(m_hbm, dstb_hbm, zeros_hbm, out_hbm,
                  didx, mrows, acc_sp, sem):
        cid = lax.axis_index("c")
        sid = lax.axis_index("s")
        wid = sid * NC + cid
        base_blk = wid * cpw
        pltpu.sync_copy(zeros_hbm.at[pl.ds(sid * rpt, rpt)],
                        acc_sp.at[pl.ds(sid * rpt, rpt)])
        plsc.subcore_barrier()

        def body(g, carry):
            blk0 = base_blk + g * gb
            pltpu.sync_copy(dstb_hbm.at[pl.ds(blk0, gb)], didx)
            pltpu.async_copy(m_hbm.at[pl.ds(blk0 * IB, gb * IB)],
                             mrows, sem).wait()
            for b in range(gb):
                pltpu.sync_copy(mrows.at[pl.ds(b * IB, IB)],
                                acc_sp.at[didx.at[b]], add=True)
            return carry

        lax.fori_loop(0, cpw // gb, body, 0)
        plsc.subcore_barrier()
        pltpu.sync_copy(acc_sp.at[pl.ds(sid * rpt, rpt)],
                        out_hbm.at[cid, pl.ds(sid * rpt, rpt)])

    return scatter_k


def _embed_body(nf_ref, w_ref, b_ref, o_ref):
    x = jnp.dot(nf_ref[...], w_ref[...],
                preferred_element_type=jnp.float32) + b_ref[...]
    o_ref[...] = jnp.concatenate([x, jnp.zeros_like(x)], axis=1)


def _edge_body(xs_ref, xd_ref, ef_ref, ws_ref, wd_ref, we_ref, b1_ref,
               w2_ref, b2_ref, m_ref):
    h = xs_ref[...].shape[1] // 2
    xs = xs_ref[...][:, :h]
    xd = xd_ref[...][:, :h]
    z = (jnp.dot(xs, ws_ref[...], preferred_element_type=jnp.float32)
         + jnp.dot(xd, wd_ref[...], preferred_element_type=jnp.float32)
         + jnp.dot(ef_ref[...], we_ref[...], preferred_element_type=jnp.float32)
         + b1_ref[...])
    z = z * jax.nn.sigmoid(z)
    m = jnp.dot(z, w2_ref[...], preferred_element_type=jnp.float32) + b2_ref[...]
    m = m * jax.nn.sigmoid(m)
    m_ref[...] = jnp.concatenate([m, jnp.zeros_like(m)], axis=1)


def _node_body(x_ref, a0_ref, a1_ref, w1x_ref, w1a_ref, b1_ref, w2_ref,
               b2_ref, o_ref):
    h = x_ref[...].shape[1] // 2
    x = x_ref[...][:, :h]
    agg = a0_ref[...][:, :h] + a1_ref[...][:, :h]
    z = (jnp.dot(x, w1x_ref[...], preferred_element_type=jnp.float32)
         + jnp.dot(agg, w1a_ref[...], preferred_element_type=jnp.float32)
         + b1_ref[...])
    z = jnp.maximum(z, 0.0)
    xo = x + jnp.dot(z, w2_ref[...], preferred_element_type=jnp.float32) \
        + b2_ref[...]
    o_ref[...] = jnp.concatenate([xo, jnp.zeros_like(xo)], axis=1)


def _unembed_body(x_ref, w1_ref, b1_ref, w2_ref, b2_ref, o_ref):
    h = x_ref[...].shape[1] // 2
    x = x_ref[...][:, :h]
    z = jnp.dot(x, w1_ref[...],
                preferred_element_type=jnp.float32) + b1_ref[...]
    z = z * jax.nn.sigmoid(z)
    o_ref[...] = jnp.dot(z, w2_ref[...],
                         preferred_element_type=jnp.float32) + b2_ref[...]


def _row_blocked(body, n_rows, blk, in_shapes, out_cols):
    grid = (n_rows // blk,)
    in_specs = []
    for s in in_shapes:
        if s[0] == n_rows:
            in_specs.append(pl.BlockSpec((blk, s[1]), lambda i: (i, 0)))
        else:
            in_specs.append(pl.BlockSpec(s, lambda i, _r=len(s): (0,) * _r))
    return pl.pallas_call(
        body,
        grid=grid,
        in_specs=in_specs,
        out_specs=pl.BlockSpec((blk, out_cols), lambda i: (i, 0)),
        out_shape=jax.ShapeDtypeStruct((n_rows, out_cols), jnp.float32),
    )


def kernel(node_features, edge_indices, edges_features, batch_size,
           emb_w, emb_b, e1_w, e1_b, e2_w, e2_b,
           n1_w, n1_b, n2_w, n2_b, u1_w, u1_b, u2_w, u2_b):
    n, df = node_features.shape
    e = edge_indices.shape[1]
    de = edges_features.shape[1]
    h = emb_w.shape[0]
    n_layers = e1_w.shape[0]
    do = u2_w.shape[0]

    n_pad = ((n + 16 + 1023) // 1024) * 1024
    cpw = -(-e // (NW * IB))
    if cpw % 2:
        cpw += 1
    gb = 2
    e_pad = NW * cpw * IB
    bn = 1024
    be = 1024

    f32 = jnp.float32
    src = edge_indices[0]
    dst = edge_indices[1]
    pad_e = e_pad - e
    pad_ids = jnp.arange(pad_e, dtype=jnp.int32)
    src_p = jnp.concatenate([src, pad_ids % n]).reshape(e_pad // IB, IB)
    dst_p = jnp.concatenate([dst, n + (pad_ids % 16)]).reshape(e_pad // IB, IB)
    ef_p = jnp.zeros((e_pad, de), f32).at[:e].set(edges_features)
    nf_p = jnp.zeros((n_pad, df), f32).at[:n].set(node_features)
    zeros_np = jnp.zeros((n_pad, W), f32)

    gather = _make_gather(n_pad, e_pad, cpw, gb)
    scatter = _make_scatter(n_pad, e_pad, cpw, gb)

    embed = _row_blocked(_embed_body, n_pad, bn,
                         [(n_pad, df), (df, h), (1, h)], W)
    edge_mlp = _row_blocked(_edge_body, e_pad, be,
                            [(e_pad, W), (e_pad, W), (e_pad, de),
                             (h, h), (h, h), (de, h), (1, h), (h, h), (1, h)],
                            W)
    node_mlp = _row_blocked(_node_body, n_pad, bn,
                            [(n_pad, W), (n_pad, W), (n_pad, W),
                             (h, h), (h, h), (1, h), (h, h), (1, h)], W)
    unembed = _row_blocked(_unembed_body, n_pad, bn,
                           [(n_pad, W), (h, h), (1, h), (h, do), (1, do)], do)

    x = embed(nf_p, emb_w.T, emb_b[None])
    for l in range(n_layers):
        ws = e1_w[l, :, :h].T
        wd = e1_w[l, :, h:2 * h].T
        we = e1_w[l, :, 2 * h:].T
        xs, xd = gather(x, src_p, dst_p)
        m = edge_mlp(xs, xd, ef_p, ws, wd, we, e1_b[l][None],
                     e2_w[l].T, e2_b[l][None])
        agg = scatter(m, dst_p, zeros_np)
        x = node_mlp(x, agg[0], agg[1], n1_w[l, :, :h].T, n1_w[l, :, h:].T,
                     n1_b[l][None], n2_w[l].T, n2_b[l][None])
    pred = unembed(x, u1_w.T, u1_b[None], u2_w.T, u2_b[None])[:n]
    return (pred, edge_indices, edges_features)


# speedup vs baseline: 2.3919x; 1.0258x over previous
import functools

import jax
import jax.numpy as jnp
from jax import lax
from jax.experimental import pallas as pl
from jax.experimental.pallas import tpu as pltpu
from jax.experimental.pallas import tpu_sc as plsc

NC = 2
NS = 16
NW = NC * NS
IB = 128
W = 128


def _mesh():
    return plsc.VectorSubcoreMesh(
        core_axis_name="c", subcore_axis_name="s", num_cores=NC,
        num_subcores=NS)


def _make_gather(n_pad, e_pad, cpw, gb):
    rpt = n_pad // NS

    @functools.partial(
        pl.kernel,
        out_type=[jax.ShapeDtypeStruct((e_pad, W), jnp.float32),
                  jax.ShapeDtypeStruct((e_pad, W), jnp.float32)],
        mesh=_mesh(),
        scratch_types=[
            pltpu.VMEM((gb, IB), jnp.int32),
            pltpu.VMEM((gb, IB), jnp.int32),
            pltpu.VMEM((gb * IB, W), jnp.float32),
            pltpu.VMEM((gb * IB, W), jnp.float32),
            pltpu.VMEM_SHARED((n_pad, W), jnp.float32),
            pltpu.SemaphoreType.DMA,
            pltpu.SemaphoreType.DMA,
        ],
    )
    def gather_k(x_hbm, srcb_hbm, dstb_hbm, xs_hbm, xd_hbm,
                 sidx, didx, srows, drows, xsp, sem_s, sem_d):
        cid = lax.axis_index("c")
        sid = lax.axis_index("s")
        wid = sid * NC + cid
        base_blk = wid * cpw
        pltpu.sync_copy(x_hbm.at[pl.ds(sid * rpt, rpt)],
                        xsp.at[pl.ds(sid * rpt, rpt)])
        plsc.subcore_barrier()

        def body(g, carry):
            blk0 = base_blk + g * gb
            pltpu.sync_copy(srcb_hbm.at[pl.ds(blk0, gb)], sidx)
            pltpu.sync_copy(dstb_hbm.at[pl.ds(blk0, gb)], didx)
            cps = [pltpu.async_copy(xsp.at[sidx.at[b]],
                                    srows.at[pl.ds(b * IB, IB)], sem_s)
                   for b in range(gb)]
            cpd = [pltpu.async_copy(xsp.at[didx.at[b]],
                                    drows.at[pl.ds(b * IB, IB)], sem_d)
                   for b in range(gb)]
            for c in cps:
                c.wait()
            for c in cpd:
                c.wait()
            pltpu.sync_copy(srows, xs_hbm.at[pl.ds(blk0 * IB, gb * IB)])
            pltpu.sync_copy(drows, xd_hbm.at[pl.ds(blk0 * IB, gb * IB)])
            return carry

        lax.fori_loop(0, cpw // gb, body, 0)

    return gather_k


def _make_scatter(n_pad, e_pad, cpw, gb):
    rpt = n_pad // NS

    @functools.partial(
        pl.kernel,
        out_type=jax.ShapeDtypeStruct((NC, n_pad, W), jnp.float32),
        mesh=_mesh(),
        scratch_types=[
            pltpu.VMEM((gb, IB), jnp.int32),
            pltpu.VMEM((gb * IB, W), jnp.float32),
            pltpu.VMEM_SHARED((n_pad, W), jnp.float32),
            pltpu.SemaphoreType.DMA,
        ],
    )
    def scatter_k(m_hbm, dstb_hbm, zeros_hbm, out_hbm,
                  didx, mrows, acc_sp, sem):
        cid = lax.axis_index("c")
        sid = lax.axis_index("s")
        wid = sid * NC + cid
        base_blk = wid * cpw
        pltpu.sync_copy(zeros_hbm.at[pl.ds(sid * rpt, rpt)],
                        acc_sp.at[pl.ds(sid * rpt, rpt)])
        plsc.subcore_barrier()

        def body(g, carry):
            blk0 = base_blk + g * gb
            pltpu.sync_copy(dstb_hbm.at[pl.ds(blk0, gb)], didx)
            pltpu.async_copy(m_hbm.at[pl.ds(blk0 * IB, gb * IB)],
                             mrows, sem).wait()
            for b in range(gb):
                pltpu.sync_copy(mrows.at[pl.ds(b * IB, IB)],
                                acc_sp.at[didx.at[b]], add=True)
            return carry

        lax.fori_loop(0, cpw // gb, body, 0)
        plsc.subcore_barrier()
        pltpu.sync_copy(acc_sp.at[pl.ds(sid * rpt, rpt)],
                        out_hbm.at[cid, pl.ds(sid * rpt, rpt)])

    return scatter_k


def _embed_body(nf_ref, w_ref, b_ref, o_ref):
    x = jnp.dot(nf_ref[...], w_ref[...],
                preferred_element_type=jnp.float32) + b_ref[...]
    o_ref[...] = jnp.concatenate([x, jnp.zeros_like(x)], axis=1)


def _edge_body(xs_ref, xd_ref, ef_ref, ws_ref, wd_ref, we_ref, b1_ref,
               w2_ref, b2_ref, m_ref):
    h = xs_ref[...].shape[1] // 2
    xs = xs_ref[...][:, :h]
    xd = xd_ref[...][:, :h]
    z = (jnp.dot(xs, ws_ref[...], preferred_element_type=jnp.float32)
         + jnp.dot(xd, wd_ref[...], preferred_element_type=jnp.float32)
         + jnp.dot(ef_ref[...], we_ref[...], preferred_element_type=jnp.float32)
         + b1_ref[...])
    z = z * jax.nn.sigmoid(z)
    m = jnp.dot(z, w2_ref[...], preferred_element_type=jnp.float32) + b2_ref[...]
    m = m * jax.nn.sigmoid(m)
    m_ref[...] = jnp.concatenate([m, jnp.zeros_like(m)], axis=1)


def _node_body(x_ref, a0_ref, a1_ref, w1x_ref, w1a_ref, b1_ref, w2_ref,
               b2_ref, o_ref):
    h = x_ref[...].shape[1] // 2
    x = x_ref[...][:, :h]
    agg = a0_ref[...][:, :h] + a1_ref[...][:, :h]
    z = (jnp.dot(x, w1x_ref[...], preferred_element_type=jnp.float32)
         + jnp.dot(agg, w1a_ref[...], preferred_element_type=jnp.float32)
         + b1_ref[...])
    z = jnp.maximum(z, 0.0)
    xo = x + jnp.dot(z, w2_ref[...], preferred_element_type=jnp.float32) \
        + b2_ref[...]
    o_ref[...] = jnp.concatenate([xo, jnp.zeros_like(xo)], axis=1)


def _unembed_body(x_ref, w1_ref, b1_ref, w2_ref, b2_ref, o_ref):
    h = x_ref[...].shape[1] // 2
    x = x_ref[...][:, :h]
    z = jnp.dot(x, w1_ref[...],
                preferred_element_type=jnp.float32) + b1_ref[...]
    z = z * jax.nn.sigmoid(z)
    o_ref[...] = jnp.dot(z, w2_ref[...],
                         preferred_element_type=jnp.float32) + b2_ref[...]


def _row_blocked(body, n_rows, blk, in_shapes, out_cols):
    grid = (n_rows // blk,)
    in_specs = []
    for s in in_shapes:
        if s[0] == n_rows:
            in_specs.append(pl.BlockSpec((blk, s[1]), lambda i: (i, 0)))
        else:
            in_specs.append(pl.BlockSpec(s, lambda i, _r=len(s): (0,) * _r))
    return pl.pallas_call(
        body,
        grid=grid,
        in_specs=in_specs,
        out_specs=pl.BlockSpec((blk, out_cols), lambda i: (i, 0)),
        out_shape=jax.ShapeDtypeStruct((n_rows, out_cols), jnp.float32),
    )


def kernel(node_features, edge_indices, edges_features, batch_size,
           emb_w, emb_b, e1_w, e1_b, e2_w, e2_b,
           n1_w, n1_b, n2_w, n2_b, u1_w, u1_b, u2_w, u2_b):
    n, df = node_features.shape
    e = edge_indices.shape[1]
    de = edges_features.shape[1]
    h = emb_w.shape[0]
    n_layers = e1_w.shape[0]
    do = u2_w.shape[0]

    n_pad = ((n + 16 + 1023) // 1024) * 1024
    cpw = -(-e // (NW * IB))
    if cpw % 2:
        cpw += 1
    gb = 2
    gb_g = 1
    e_pad = NW * cpw * IB
    bn = 1024
    be = 1024

    f32 = jnp.float32
    src = edge_indices[0]
    dst = edge_indices[1]
    pad_e = e_pad - e
    pad_ids = jnp.arange(pad_e, dtype=jnp.int32)
    src_p = jnp.concatenate([src, pad_ids % n]).reshape(e_pad // IB, IB)
    dst_p = jnp.concatenate([dst, n + (pad_ids % 16)]).reshape(e_pad // IB, IB)
    ef_p = jnp.zeros((e_pad, de), f32).at[:e].set(edges_features)
    nf_p = jnp.zeros((n_pad, df), f32).at[:n].set(node_features)
    zeros_np = jnp.zeros((n_pad, W), f32)

    gather = _make_gather(n_pad, e_pad, cpw, gb_g)
    scatter = _make_scatter(n_pad, e_pad, cpw, gb)

    embed = _row_blocked(_embed_body, n_pad, bn,
                         [(n_pad, df), (df, h), (1, h)], W)
    edge_mlp = _row_blocked(_edge_body, e_pad, be,
                            [(e_pad, W), (e_pad, W), (e_pad, de),
                             (h, h), (h, h), (de, h), (1, h), (h, h), (1, h)],
                            W)
    node_mlp = _row_blocked(_node_body, n_pad, bn,
                            [(n_pad, W), (n_pad, W), (n_pad, W),
                             (h, h), (h, h), (1, h), (h, h), (1, h)], W)
    unembed = _row_blocked(_unembed_body, n_pad, bn,
                           [(n_pad, W), (h, h), (1, h), (h, do), (1, do)], do)

    x = embed(nf_p, emb_w.T, emb_b[None])
    for l in range(n_layers):
        ws = e1_w[l, :, :h].T
        wd = e1_w[l, :, h:2 * h].T
        we = e1_w[l, :, 2 * h:].T
        xs, xd = gather(x, src_p, dst_p)
        m = edge_mlp(xs, xd, ef_p, ws, wd, we, e1_b[l][None],
                     e2_w[l].T, e2_b[l][None])
        agg = scatter(m, dst_p, zeros_np)
        x = node_mlp(x, agg[0], agg[1], n1_w[l, :, :h].T, n1_w[l, :, h:].T,
                     n1_b[l][None], n2_w[l].T, n2_b[l][None])
    pred = unembed(x, u1_w.T, u1_b[None], u2_w.T, u2_b[None])[:n]
    return (pred, edge_indices, edges_features)


# speedup vs baseline: 2.4015x; 1.0040x over previous
import functools

import jax
import jax.numpy as jnp
from jax import lax
from jax.experimental import pallas as pl
from jax.experimental.pallas import tpu as pltpu
from jax.experimental.pallas import tpu_sc as plsc

NC = 2
NS = 16
NW = NC * NS
IB = 128
W = 128


def _mesh():
    return plsc.VectorSubcoreMesh(
        core_axis_name="c", subcore_axis_name="s", num_cores=NC,
        num_subcores=NS)


def _make_gather(n_pad, e_pad, cpw, gb):
    del gb

    @functools.partial(
        pl.kernel,
        out_type=[jax.ShapeDtypeStruct((e_pad, W), jnp.float32),
                  jax.ShapeDtypeStruct((e_pad, W), jnp.float32)],
        mesh=_mesh(),
        scratch_types=[
            pltpu.VMEM((2, IB), jnp.int32),
            pltpu.VMEM((2, IB), jnp.int32),
            [pltpu.VMEM((IB, W), jnp.float32) for _ in range(2)],
            [pltpu.VMEM((IB, W), jnp.float32) for _ in range(2)],
            pltpu.SemaphoreType.DMA,
            pltpu.SemaphoreType.DMA,
            [pltpu.SemaphoreType.DMA for _ in range(2)],
            [pltpu.SemaphoreType.DMA for _ in range(2)],
        ],
    )
    def gather_k(x_hbm, srcb_hbm, dstb_hbm, xs_hbm, xd_hbm,
                 sidx, didx, srows, drows, sem_s, sem_d, semw_s, semw_d):
        cid = lax.axis_index("c")
        sid = lax.axis_index("s")
        wid = sid * NC + cid
        base_blk = wid * cpw

        def load_pair(gg):
            pltpu.sync_copy(srcb_hbm.at[pl.ds(base_blk + 2 * gg, 2)], sidx)
            pltpu.sync_copy(dstb_hbm.at[pl.ds(base_blk + 2 * gg, 2)], didx)

        def fire_gather(g, buf):
            pltpu.async_copy(x_hbm.at[sidx.at[buf]], srows[buf], sem_s)
            pltpu.async_copy(x_hbm.at[didx.at[buf]], drows[buf], sem_d)

        def wait_gather(buf):
            pltpu.make_async_copy(x_hbm.at[sidx.at[buf]], srows[buf],
                                  sem_s).wait()
            pltpu.make_async_copy(x_hbm.at[didx.at[buf]], drows[buf],
                                  sem_d).wait()

        def fire_write(g, buf):
            row0 = (base_blk + g) * IB
            pltpu.async_copy(srows[buf], xs_hbm.at[pl.ds(row0, IB)],
                             semw_s[buf])
            pltpu.async_copy(drows[buf], xd_hbm.at[pl.ds(row0, IB)],
                             semw_d[buf])

        def wait_write(g, buf):
            row0 = (base_blk + g) * IB
            pltpu.make_async_copy(srows[buf], xs_hbm.at[pl.ds(row0, IB)],
                                  semw_s[buf]).wait()
            pltpu.make_async_copy(drows[buf], xd_hbm.at[pl.ds(row0, IB)],
                                  semw_d[buf]).wait()

        load_pair(0)
        fire_gather(0, 0)

        def body(gg, carry):
            for half in range(2):
                g = 2 * gg + half
                buf = half
                nbuf = 1 - half
                wait_gather(buf)

                @pl.when(g + 1 < cpw)
                def _():
                    @pl.when(g >= 1)
                    def _():
                        wait_write(g - 1, nbuf)
                    if half == 1:
                        load_pair(gg + 1)
                    fire_gather(g + 1, nbuf)

                fire_write(g, buf)
            return carry

        lax.fori_loop(0, cpw // 2, body, 0)
        wait_write(cpw - 2, 0)
        wait_write(cpw - 1, 1)

    return gather_k


def _make_scatter(n_pad, e_pad, cpw, gb):
    rpt = n_pad // NS

    @functools.partial(
        pl.kernel,
        out_type=jax.ShapeDtypeStruct((NC, n_pad, W), jnp.float32),
        mesh=_mesh(),
        scratch_types=[
            pltpu.VMEM((gb, IB), jnp.int32),
            pltpu.VMEM((gb * IB, W), jnp.float32),
            pltpu.VMEM_SHARED((n_pad, W), jnp.float32),
            pltpu.SemaphoreType.DMA,
        ],
    )
    def scatter_k(m_hbm, dstb_hbm, zeros_hbm, out_hbm,
                  didx, mrows, acc_sp, sem):
        cid = lax.axis_index("c")
        sid = lax.axis_index("s")
        wid = sid * NC + cid
        base_blk = wid * cpw
        pltpu.sync_copy(zeros_hbm.at[pl.ds(sid * rpt, rpt)],
                        acc_sp.at[pl.ds(sid * rpt, rpt)])
        plsc.subcore_barrier()

        def body(g, carry):
            blk0 = base_blk + g * gb
            pltpu.sync_copy(dstb_hbm.at[pl.ds(blk0, gb)], didx)
            pltpu.async_copy(m_hbm.at[pl.ds(blk0 * IB, gb * IB)],
                             mrows, sem).wait()
            for b in range(gb):
                pltpu.sync_copy(mrows.at[pl.ds(b * IB, IB)],
                                acc_sp.at[didx.at[b]], add=True)
            return carry

        lax.fori_loop(0, cpw // gb, body, 0)
        plsc.subcore_barrier()
        pltpu.sync_copy(acc_sp.at[pl.ds(sid * rpt, rpt)],
                        out_hbm.at[cid, pl.ds(sid * rpt, rpt)])

    return scatter_k


def _embed_body(nf_ref, w_ref, b_ref, o_ref):
    x = jnp.dot(nf_ref[...], w_ref[...],
                preferred_element_type=jnp.float32) + b_ref[...]
    o_ref[...] = jnp.concatenate([x, jnp.zeros_like(x)], axis=1)


def _edge_body(xs_ref, xd_ref, ef_ref, ws_ref, wd_ref, we_ref, b1_ref,
               w2_ref, b2_ref, m_ref):
    h = xs_ref[...].shape[1] // 2
    xs = xs_ref[...][:, :h]
    xd = xd_ref[...][:, :h]
    z = (jnp.dot(xs, ws_ref[...], preferred_element_type=jnp.float32)
         + jnp.dot(xd, wd_ref[...], preferred_element_type=jnp.float32)
         + jnp.dot(ef_ref[...], we_ref[...], preferred_element_type=jnp.float32)
         + b1_ref[...])
    z = z * jax.nn.sigmoid(z)
    m = jnp.dot(z, w2_ref[...], preferred_element_type=jnp.float32) + b2_ref[...]
    m = m * jax.nn.sigmoid(m)
    m_ref[...] = jnp.concatenate([m, jnp.zeros_like(m)], axis=1)


def _node_body(x_ref, a0_ref, a1_ref, w1x_ref, w1a_ref, b1_ref, w2_ref,
               b2_ref, o_ref):
    h = x_ref[...].shape[1] // 2
    x = x_ref[...][:, :h]
    agg = a0_ref[...][:, :h] + a1_ref[...][:, :h]
    z = (jnp.dot(x, w1x_ref[...], preferred_element_type=jnp.float32)
         + jnp.dot(agg, w1a_ref[...], preferred_element_type=jnp.float32)
         + b1_ref[...])
    z = jnp.maximum(z, 0.0)
    xo = x + jnp.dot(z, w2_ref[...], preferred_element_type=jnp.float32) \
        + b2_ref[...]
    o_ref[...] = jnp.concatenate([xo, jnp.zeros_like(xo)], axis=1)


def _unembed_body(x_ref, w1_ref, b1_ref, w2_ref, b2_ref, o_ref):
    h = x_ref[...].shape[1] // 2
    x = x_ref[...][:, :h]
    z = jnp.dot(x, w1_ref[...],
                preferred_element_type=jnp.float32) + b1_ref[...]
    z = z * jax.nn.sigmoid(z)
    o_ref[...] = jnp.dot(z, w2_ref[...],
                         preferred_element_type=jnp.float32) + b2_ref[...]


def _row_blocked(body, n_rows, blk, in_shapes, out_cols,
                 out_dtype=jnp.float32):
    grid = (n_rows // blk,)
    in_specs = []
    for s in in_shapes:
        if s[0] == n_rows:
            in_specs.append(pl.BlockSpec((blk, s[1]), lambda i: (i, 0)))
        else:
            in_specs.append(pl.BlockSpec(s, lambda i, _r=len(s): (0,) * _r))
    return pl.pallas_call(
        body,
        grid=grid,
        in_specs=in_specs,
        out_specs=pl.BlockSpec((blk, out_cols), lambda i: (i, 0)),
        out_shape=jax.ShapeDtypeStruct((n_rows, out_cols), out_dtype),
    )


def kernel(node_features, edge_indices, edges_features, batch_size,
           emb_w, emb_b, e1_w, e1_b, e2_w, e2_b,
           n1_w, n1_b, n2_w, n2_b, u1_w, u1_b, u2_w, u2_b):
    n, df = node_features.shape
    e = edge_indices.shape[1]
    de = edges_features.shape[1]
    h = emb_w.shape[0]
    n_layers = e1_w.shape[0]
    do = u2_w.shape[0]

    n_pad = ((n + 16 + 1023) // 1024) * 1024
    cpw = -(-e // (NW * IB))
    if cpw % 2:
        cpw += 1
    gb = 2
    gb_g = 1
    e_pad = NW * cpw * IB
    bn = 1024
    be = 1024

    f32 = jnp.float32
    src = edge_indices[0]
    dst = edge_indices[1]
    pad_e = e_pad - e
    pad_ids = jnp.arange(pad_e, dtype=jnp.int32)
    src_p = jnp.concatenate([src, pad_ids % n]).reshape(e_pad // IB, IB)
    dst_p = jnp.concatenate([dst, n + (pad_ids % 16)]).reshape(e_pad // IB, IB)
    ef_p = jnp.zeros((e_pad, de), f32).at[:e].set(edges_features)
    nf_p = jnp.zeros((n_pad, df), f32).at[:n].set(node_features)
    zeros_np = jnp.zeros((n_pad, W), f32)

    gather = _make_gather(n_pad, e_pad, cpw, gb_g)
    scatter = _make_scatter(n_pad, e_pad, cpw, gb)

    embed = _row_blocked(_embed_body, n_pad, bn,
                         [(n_pad, df), (df, h), (1, h)], W)
    edge_mlp = _row_blocked(_edge_body, e_pad, be,
                            [(e_pad, W), (e_pad, W), (e_pad, de),
                             (h, h), (h, h), (de, h), (1, h), (h, h), (1, h)],
                            W)
    node_mlp = _row_blocked(_node_body, n_pad, bn,
                            [(n_pad, W), (n_pad, W), (n_pad, W),
                             (h, h), (h, h), (1, h), (h, h), (1, h)], W)
    unembed = _row_blocked(_unembed_body, n_pad, bn,
                           [(n_pad, W), (h, h), (1, h), (h, do), (1, do)], do)

    x = embed(nf_p, emb_w.T, emb_b[None])
    for l in range(n_layers):
        ws = e1_w[l, :, :h].T
        wd = e1_w[l, :, h:2 * h].T
        we = e1_w[l, :, 2 * h:].T
        xs, xd = gather(x, src_p, dst_p)
        m = edge_mlp(xs, xd, ef_p, ws, wd, we, e1_b[l][None],
                     e2_w[l].T, e2_b[l][None])
        agg = scatter(m, dst_p, zeros_np)
        x = node_mlp(x, agg[0], agg[1], n1_w[l, :, :h].T, n1_w[l, :, h:].T,
                     n1_b[l][None], n2_w[l].T, n2_b[l][None])
    pred = unembed(x, u1_w.T, u1_b[None], u2_w.T, u2_b[None])[:n]
    return (pred, edge_indices, edges_features)


# speedup vs baseline: 2.7543x; 1.1469x over previous
import functools

import jax
import jax.numpy as jnp
from jax import lax
from jax.experimental import pallas as pl
from jax.experimental.pallas import tpu as pltpu
from jax.experimental.pallas import tpu_sc as plsc

NC = 2
NS = 16
NW = NC * NS
IB = 128
W = 128


def _mesh():
    return plsc.VectorSubcoreMesh(
        core_axis_name="c", subcore_axis_name="s", num_cores=NC,
        num_subcores=NS)


def _make_gather(n_pad, e_pad, cpw, gb):
    del gb

    @functools.partial(
        pl.kernel,
        out_type=[jax.ShapeDtypeStruct((e_pad, W), jnp.float32),
                  jax.ShapeDtypeStruct((e_pad, W), jnp.float32)],
        mesh=_mesh(),
        scratch_types=[
            pltpu.VMEM((2, IB), jnp.int32),
            pltpu.VMEM((2, IB), jnp.int32),
            [pltpu.VMEM((IB, W), jnp.float32) for _ in range(2)],
            [pltpu.VMEM((IB, W), jnp.float32) for _ in range(2)],
            pltpu.SemaphoreType.DMA,
            pltpu.SemaphoreType.DMA,
            [pltpu.SemaphoreType.DMA for _ in range(2)],
            [pltpu.SemaphoreType.DMA for _ in range(2)],
        ],
    )
    def gather_k(x_hbm, srcb_hbm, dstb_hbm, xs_hbm, xd_hbm,
                 sidx, didx, srows, drows, sem_s, sem_d, semw_s, semw_d):
        cid = lax.axis_index("c")
        sid = lax.axis_index("s")
        wid = sid * NC + cid
        base_blk = wid * cpw

        def load_pair(gg):
            pltpu.sync_copy(srcb_hbm.at[pl.ds(base_blk + 2 * gg, 2)], sidx)
            pltpu.sync_copy(dstb_hbm.at[pl.ds(base_blk + 2 * gg, 2)], didx)

        def fire_gather(g, buf):
            pltpu.async_copy(x_hbm.at[sidx.at[buf]], srows[buf], sem_s)
            pltpu.async_copy(x_hbm.at[didx.at[buf]], drows[buf], sem_d)

        def wait_gather(buf):
            pltpu.make_async_copy(x_hbm.at[sidx.at[buf]], srows[buf],
                                  sem_s).wait()
            pltpu.make_async_copy(x_hbm.at[didx.at[buf]], drows[buf],
                                  sem_d).wait()

        def fire_write(g, buf):
            row0 = (base_blk + g) * IB
            pltpu.async_copy(srows[buf], xs_hbm.at[pl.ds(row0, IB)],
                             semw_s[buf])
            pltpu.async_copy(drows[buf], xd_hbm.at[pl.ds(row0, IB)],
                             semw_d[buf])

        def wait_write(g, buf):
            row0 = (base_blk + g) * IB
            pltpu.make_async_copy(srows[buf], xs_hbm.at[pl.ds(row0, IB)],
                                  semw_s[buf]).wait()
            pltpu.make_async_copy(drows[buf], xd_hbm.at[pl.ds(row0, IB)],
                                  semw_d[buf]).wait()

        load_pair(0)
        fire_gather(0, 0)

        def body(gg, carry):
            for half in range(2):
                g = 2 * gg + half
                buf = half
                nbuf = 1 - half
                wait_gather(buf)

                @pl.when(g + 1 < cpw)
                def _():
                    @pl.when(g >= 1)
                    def _():
                        wait_write(g - 1, nbuf)
                    if half == 1:
                        load_pair(gg + 1)
                    fire_gather(g + 1, nbuf)

                fire_write(g, buf)
            return carry

        lax.fori_loop(0, cpw // 2, body, 0)
        wait_write(cpw - 2, 0)
        wait_write(cpw - 1, 1)

    return gather_k


def _make_scatter(n_pad, e_pad, cpw, gb):
    rpt = n_pad // NS

    @functools.partial(
        pl.kernel,
        out_type=jax.ShapeDtypeStruct((NC, n_pad, W), jnp.float32),
        mesh=_mesh(),
        scratch_types=[
            pltpu.VMEM((gb, IB), jnp.int32),
            pltpu.VMEM((gb * IB, W), jnp.float32),
            pltpu.VMEM_SHARED((n_pad, W), jnp.float32),
            pltpu.SemaphoreType.DMA,
        ],
    )
    def scatter_k(m_hbm, dstb_hbm, zeros_hbm, out_hbm,
                  didx, mrows, acc_sp, sem):
        cid = lax.axis_index("c")
        sid = lax.axis_index("s")
        wid = sid * NC + cid
        base_blk = wid * cpw
        pltpu.sync_copy(zeros_hbm.at[pl.ds(sid * rpt, rpt)],
                        acc_sp.at[pl.ds(sid * rpt, rpt)])
        plsc.subcore_barrier()

        def body(g, carry):
            blk0 = base_blk + g * gb
            pltpu.sync_copy(dstb_hbm.at[pl.ds(blk0, gb)], didx)
            pltpu.async_copy(m_hbm.at[pl.ds(blk0 * IB, gb * IB)],
                             mrows, sem).wait()
            for b in range(gb):
                pltpu.sync_copy(mrows.at[pl.ds(b * IB, IB)],
                                acc_sp.at[didx.at[b]], add=True)
            return carry

        lax.fori_loop(0, cpw // gb, body, 0)
        plsc.subcore_barrier()
        pltpu.sync_copy(acc_sp.at[pl.ds(sid * rpt, rpt)],
                        out_hbm.at[cid, pl.ds(sid * rpt, rpt)])

    return scatter_k


def _embed_body(nf_ref, w_ref, b_ref, o_ref):
    x = jnp.dot(nf_ref[...], w_ref[...],
                preferred_element_type=jnp.float32) + b_ref[...]
    o_ref[...] = jnp.concatenate([x, jnp.zeros_like(x)], axis=1)


def _edge_body(xs_ref, xd_ref, ef_ref, ws_ref, wd_ref, we_ref, b1_ref,
               w2_ref, b2_ref, m_ref):
    h = xs_ref[...].shape[1] // 2
    xs = xs_ref[...][:, :h]
    xd = xd_ref[...][:, :h]
    z = (jnp.dot(xs, ws_ref[...], preferred_element_type=jnp.float32)
         + jnp.dot(xd, wd_ref[...], preferred_element_type=jnp.float32)
         + jnp.dot(ef_ref[...], we_ref[...], preferred_element_type=jnp.float32)
         + b1_ref[...])
    z = z * jax.nn.sigmoid(z)
    m = jnp.dot(z, w2_ref[...], preferred_element_type=jnp.float32) + b2_ref[...]
    m = m * jax.nn.sigmoid(m)
    m_ref[...] = jnp.concatenate([m, jnp.zeros_like(m)], axis=1)


def _node_body(x_ref, a0_ref, a1_ref, a2_ref, a3_ref, w1x_ref, w1a_ref,
               b1_ref, w2_ref, b2_ref, o_ref):
    h = x_ref[...].shape[1] // 2
    x = x_ref[...][:, :h]
    agg = (a0_ref[...][:, :h] + a1_ref[...][:, :h]
           + a2_ref[...][:, :h] + a3_ref[...][:, :h])
    z = (jnp.dot(x, w1x_ref[...], preferred_element_type=jnp.float32)
         + jnp.dot(agg, w1a_ref[...], preferred_element_type=jnp.float32)
         + b1_ref[...])
    z = jnp.maximum(z, 0.0)
    xo = x + jnp.dot(z, w2_ref[...], preferred_element_type=jnp.float32) \
        + b2_ref[...]
    o_ref[...] = jnp.concatenate([xo, jnp.zeros_like(xo)], axis=1)


def _unembed_body(x_ref, w1_ref, b1_ref, w2_ref, b2_ref, o_ref):
    h = x_ref[...].shape[1] // 2
    x = x_ref[...][:, :h]
    z = jnp.dot(x, w1_ref[...],
                preferred_element_type=jnp.float32) + b1_ref[...]
    z = z * jax.nn.sigmoid(z)
    o_ref[...] = jnp.dot(z, w2_ref[...],
                         preferred_element_type=jnp.float32) + b2_ref[...]


def _row_blocked(body, n_rows, blk, in_shapes, out_cols,
                 out_dtype=jnp.float32):
    grid = (n_rows // blk,)
    in_specs = []
    for s in in_shapes:
        if s[0] == n_rows:
            in_specs.append(pl.BlockSpec((blk, s[1]), lambda i: (i, 0)))
        else:
            in_specs.append(pl.BlockSpec(s, lambda i, _r=len(s): (0,) * _r))
    return pl.pallas_call(
        body,
        grid=grid,
        in_specs=in_specs,
        out_specs=pl.BlockSpec((blk, out_cols), lambda i: (i, 0)),
        out_shape=jax.ShapeDtypeStruct((n_rows, out_cols), out_dtype),
    )


def kernel(node_features, edge_indices, edges_features, batch_size,
           emb_w, emb_b, e1_w, e1_b, e2_w, e2_b,
           n1_w, n1_b, n2_w, n2_b, u1_w, u1_b, u2_w, u2_b):
    n, df = node_features.shape
    e = edge_indices.shape[1]
    de = edges_features.shape[1]
    h = emb_w.shape[0]
    n_layers = e1_w.shape[0]
    do = u2_w.shape[0]

    n_pad = ((n + 16 + 1023) // 1024) * 1024
    cpw = -(-e // (NW * IB))
    if cpw % 2:
        cpw += 1
    gb = 2
    gb_g = 1
    e_pad = NW * cpw * IB
    bn = 1024
    be = 1024

    f32 = jnp.float32
    src = edge_indices[0]
    dst = edge_indices[1]
    pad_e = e_pad - e
    pad_ids = jnp.arange(pad_e, dtype=jnp.int32)
    src_p = jnp.concatenate([src, pad_ids % n]).reshape(e_pad // IB, IB)
    dst_p = jnp.concatenate([dst, n + (pad_ids % 16)]).reshape(e_pad // IB, IB)
    ef_p = jnp.zeros((e_pad, de), f32).at[:e].set(edges_features)
    nf_p = jnp.zeros((n_pad, df), f32).at[:n].set(node_features)
    zeros_np = jnp.zeros((n_pad, W), f32)

    nck = 2
    cpw_c = cpw // nck
    e_chk = e_pad // nck
    nblk_c = e_chk // IB
    gather = _make_gather(n_pad, e_chk, cpw_c, gb_g)
    scatter = _make_scatter(n_pad, e_chk, cpw_c, gb)
    srcb = [src_p[c * nblk_c:(c + 1) * nblk_c] for c in range(nck)]
    dstb = [dst_p[c * nblk_c:(c + 1) * nblk_c] for c in range(nck)]
    efc = [ef_p[c * e_chk:(c + 1) * e_chk] for c in range(nck)]

    embed = _row_blocked(_embed_body, n_pad, bn,
                         [(n_pad, df), (df, h), (1, h)], W)
    edge_mlp = _row_blocked(_edge_body, e_chk, be,
                            [(e_chk, W), (e_chk, W), (e_chk, de),
                             (h, h), (h, h), (de, h), (1, h), (h, h), (1, h)],
                            W)
    node_mlp = _row_blocked(_node_body, n_pad, bn,
                            [(n_pad, W), (n_pad, W), (n_pad, W),
                             (n_pad, W), (n_pad, W),
                             (h, h), (h, h), (1, h), (h, h), (1, h)], W)
    unembed = _row_blocked(_unembed_body, n_pad, bn,
                           [(n_pad, W), (h, h), (1, h), (h, do), (1, do)], do)

    x = embed(nf_p, emb_w.T, emb_b[None])
    for l in range(n_layers):
        ws = e1_w[l, :, :h].T
        wd = e1_w[l, :, h:2 * h].T
        we = e1_w[l, :, 2 * h:].T
        aggs = []
        for c in range(nck):
            xs, xd = gather(x, srcb[c], dstb[c])
            m = edge_mlp(xs, xd, efc[c], ws, wd, we, e1_b[l][None],
                         e2_w[l].T, e2_b[l][None])
            aggs.append(scatter(m, dstb[c], zeros_np))
        x = node_mlp(x, aggs[0][0], aggs[0][1], aggs[1][0], aggs[1][1],
                     n1_w[l, :, :h].T, n1_w[l, :, h:].T,
                     n1_b[l][None], n2_w[l].T, n2_b[l][None])
    pred = unembed(x, u1_w.T, u1_b[None], u2_w.T, u2_b[None])[:n]
    return (pred, edge_indices, edges_features)


# speedup vs baseline: 2.8536x; 1.0360x over previous
import functools

import jax
import jax.numpy as jnp
from jax import lax
from jax.experimental import pallas as pl
from jax.experimental.pallas import tpu as pltpu
from jax.experimental.pallas import tpu_sc as plsc

NC = 2
NS = 16
NW = NC * NS
IB = 128
W = 128


def _mesh():
    return plsc.VectorSubcoreMesh(
        core_axis_name="c", subcore_axis_name="s", num_cores=NC,
        num_subcores=NS)


def _make_gather(n_pad, e_pad, cpw, gb):
    del gb

    @functools.partial(
        pl.kernel,
        out_type=[jax.ShapeDtypeStruct((e_pad, W), jnp.float32),
                  jax.ShapeDtypeStruct((e_pad, W), jnp.float32)],
        mesh=_mesh(),
        scratch_types=[
            pltpu.VMEM((cpw, IB), jnp.int32),
            pltpu.VMEM((cpw, IB), jnp.int32),
            [pltpu.VMEM((IB, W), jnp.float32) for _ in range(2)],
            [pltpu.VMEM((IB, W), jnp.float32) for _ in range(2)],
            pltpu.SemaphoreType.DMA,
            pltpu.SemaphoreType.DMA,
            [pltpu.SemaphoreType.DMA for _ in range(2)],
            [pltpu.SemaphoreType.DMA for _ in range(2)],
        ],
    )
    def gather_k(x_hbm, srcb_hbm, dstb_hbm, xs_hbm, xd_hbm,
                 sidx, didx, srows, drows, sem_s, sem_d, semw_s, semw_d):
        cid = lax.axis_index("c")
        sid = lax.axis_index("s")
        wid = sid * NC + cid
        base_blk = wid * cpw
        pltpu.sync_copy(srcb_hbm.at[pl.ds(base_blk, cpw)], sidx)
        pltpu.sync_copy(dstb_hbm.at[pl.ds(base_blk, cpw)], didx)

        def fire_gather(g, buf):
            pltpu.async_copy(x_hbm.at[sidx.at[g]], srows[buf], sem_s)
            pltpu.async_copy(x_hbm.at[didx.at[g]], drows[buf], sem_d)

        def wait_gather(buf):
            pltpu.make_async_copy(x_hbm.at[sidx.at[0]], srows[buf],
                                  sem_s).wait()
            pltpu.make_async_copy(x_hbm.at[didx.at[0]], drows[buf],
                                  sem_d).wait()

        def fire_write(g, buf):
            row0 = (base_blk + g) * IB
            pltpu.async_copy(srows[buf], xs_hbm.at[pl.ds(row0, IB)],
                             semw_s[buf])
            pltpu.async_copy(drows[buf], xd_hbm.at[pl.ds(row0, IB)],
                             semw_d[buf])

        def wait_write(g, buf):
            row0 = (base_blk + g) * IB
            pltpu.make_async_copy(srows[buf], xs_hbm.at[pl.ds(row0, IB)],
                                  semw_s[buf]).wait()
            pltpu.make_async_copy(drows[buf], xd_hbm.at[pl.ds(row0, IB)],
                                  semw_d[buf]).wait()

        fire_gather(0, 0)

        def body(gg, carry):
            for half in range(2):
                g = 2 * gg + half
                buf = half
                nbuf = 1 - half
                wait_gather(buf)

                @pl.when(g + 1 < cpw)
                def _():
                    @pl.when(g >= 1)
                    def _():
                        wait_write(g - 1, nbuf)
                    fire_gather(g + 1, nbuf)

                fire_write(g, buf)
            return carry

        lax.fori_loop(0, cpw // 2, body, 0)
        wait_write(cpw - 2, 0)
        wait_write(cpw - 1, 1)

    return gather_k


def _make_scatter(n_pad, e_pad, cpw, gb):
    rpt = n_pad // NS

    @functools.partial(
        pl.kernel,
        out_type=jax.ShapeDtypeStruct((NC, n_pad, W), jnp.float32),
        mesh=_mesh(),
        scratch_types=[
            pltpu.VMEM((cpw, IB), jnp.int32),
            [pltpu.VMEM((IB, W), jnp.float32) for _ in range(2)],
            pltpu.VMEM_SHARED((n_pad, W), jnp.float32),
            [pltpu.SemaphoreType.DMA for _ in range(2)],
        ],
    )
    def scatter_k(m_hbm, dstb_hbm, zeros_hbm, out_hbm,
                  didx, mrows, acc_sp, sem):
        cid = lax.axis_index("c")
        sid = lax.axis_index("s")
        wid = sid * NC + cid
        base_blk = wid * cpw
        pltpu.sync_copy(dstb_hbm.at[pl.ds(base_blk, cpw)], didx)
        pltpu.sync_copy(zeros_hbm.at[pl.ds(sid * rpt, rpt)],
                        acc_sp.at[pl.ds(sid * rpt, rpt)])
        plsc.subcore_barrier()

        def fire_read(g, buf):
            pltpu.async_copy(m_hbm.at[pl.ds((base_blk + g) * IB, IB)],
                             mrows[buf], sem[buf])

        def wait_read(buf):
            pltpu.make_async_copy(m_hbm.at[pl.ds(base_blk * IB, IB)],
                                  mrows[buf], sem[buf]).wait()

        fire_read(0, 0)

        def body(gg, carry):
            for half in range(2):
                g = 2 * gg + half
                buf = half
                wait_read(buf)

                @pl.when(g + 1 < cpw)
                def _():
                    fire_read(g + 1, 1 - half)

                pltpu.sync_copy(mrows[buf], acc_sp.at[didx.at[g]], add=True)
            return carry

        lax.fori_loop(0, cpw // 2, body, 0)
        plsc.subcore_barrier()
        pltpu.sync_copy(acc_sp.at[pl.ds(sid * rpt, rpt)],
                        out_hbm.at[cid, pl.ds(sid * rpt, rpt)])

    return scatter_k


def _embed_body(nf_ref, w_ref, b_ref, o_ref):
    x = jnp.dot(nf_ref[...], w_ref[...],
                preferred_element_type=jnp.float32) + b_ref[...]
    o_ref[...] = jnp.concatenate([x, jnp.zeros_like(x)], axis=1)


def _edge_body(xs_ref, xd_ref, ef_ref, ws_ref, wd_ref, we_ref, b1_ref,
               w2_ref, b2_ref, m_ref):
    h = xs_ref[...].shape[1] // 2
    xs = xs_ref[...][:, :h]
    xd = xd_ref[...][:, :h]
    z = (jnp.dot(xs, ws_ref[...], preferred_element_type=jnp.float32)
         + jnp.dot(xd, wd_ref[...], preferred_element_type=jnp.float32)
         + jnp.dot(ef_ref[...], we_ref[...], preferred_element_type=jnp.float32)
         + b1_ref[...])
    z = z * jax.nn.sigmoid(z)
    m = jnp.dot(z, w2_ref[...], preferred_element_type=jnp.float32) + b2_ref[...]
    m = m * jax.nn.sigmoid(m)
    m_ref[...] = jnp.concatenate([m, jnp.zeros_like(m)], axis=1)


def _node_body(x_ref, a0_ref, a1_ref, a2_ref, a3_ref, w1x_ref, w1a_ref,
               b1_ref, w2_ref, b2_ref, o_ref):
    h = x_ref[...].shape[1] // 2
    x = x_ref[...][:, :h]
    agg = (a0_ref[...][:, :h] + a1_ref[...][:, :h]
           + a2_ref[...][:, :h] + a3_ref[...][:, :h])
    z = (jnp.dot(x, w1x_ref[...], preferred_element_type=jnp.float32)
         + jnp.dot(agg, w1a_ref[...], preferred_element_type=jnp.float32)
         + b1_ref[...])
    z = jnp.maximum(z, 0.0)
    xo = x + jnp.dot(z, w2_ref[...], preferred_element_type=jnp.float32) \
        + b2_ref[...]
    o_ref[...] = jnp.concatenate([xo, jnp.zeros_like(xo)], axis=1)


def _unembed_body(x_ref, w1_ref, b1_ref, w2_ref, b2_ref, o_ref):
    h = x_ref[...].shape[1] // 2
    x = x_ref[...][:, :h]
    z = jnp.dot(x, w1_ref[...],
                preferred_element_type=jnp.float32) + b1_ref[...]
    z = z * jax.nn.sigmoid(z)
    o_ref[...] = jnp.dot(z, w2_ref[...],
                         preferred_element_type=jnp.float32) + b2_ref[...]


def _row_blocked(body, n_rows, blk, in_shapes, out_cols,
                 out_dtype=jnp.float32):
    grid = (n_rows // blk,)
    in_specs = []
    for s in in_shapes:
        if s[0] == n_rows:
            in_specs.append(pl.BlockSpec((blk, s[1]), lambda i: (i, 0)))
        else:
            in_specs.append(pl.BlockSpec(s, lambda i, _r=len(s): (0,) * _r))
    return pl.pallas_call(
        body,
        grid=grid,
        in_specs=in_specs,
        out_specs=pl.BlockSpec((blk, out_cols), lambda i: (i, 0)),
        out_shape=jax.ShapeDtypeStruct((n_rows, out_cols), out_dtype),
    )


def kernel(node_features, edge_indices, edges_features, batch_size,
           emb_w, emb_b, e1_w, e1_b, e2_w, e2_b,
           n1_w, n1_b, n2_w, n2_b, u1_w, u1_b, u2_w, u2_b):
    n, df = node_features.shape
    e = edge_indices.shape[1]
    de = edges_features.shape[1]
    h = emb_w.shape[0]
    n_layers = e1_w.shape[0]
    do = u2_w.shape[0]

    n_pad = ((n + 16 + 1023) // 1024) * 1024
    cpw = -(-e // (NW * IB))
    if cpw % 2:
        cpw += 1
    gb = 2
    gb_g = 1
    e_pad = NW * cpw * IB
    bn = 1024
    be = 1024

    f32 = jnp.float32
    src = edge_indices[0]
    dst = edge_indices[1]
    pad_e = e_pad - e
    pad_ids = jnp.arange(pad_e, dtype=jnp.int32)
    src_p = jnp.concatenate([src, pad_ids % n]).reshape(e_pad // IB, IB)
    dst_p = jnp.concatenate([dst, n + (pad_ids % 16)]).reshape(e_pad // IB, IB)
    ef_p = jnp.zeros((e_pad, de), f32).at[:e].set(edges_features)
    nf_p = jnp.zeros((n_pad, df), f32).at[:n].set(node_features)
    zeros_np = jnp.zeros((n_pad, W), f32)

    nck = 2
    cpw_c = cpw // nck
    e_chk = e_pad // nck
    nblk_c = e_chk // IB
    gather = _make_gather(n_pad, e_chk, cpw_c, gb_g)
    scatter = _make_scatter(n_pad, e_chk, cpw_c, gb)
    srcb = [src_p[c * nblk_c:(c + 1) * nblk_c] for c in range(nck)]
    dstb = [dst_p[c * nblk_c:(c + 1) * nblk_c] for c in range(nck)]
    efc = [ef_p[c * e_chk:(c + 1) * e_chk] for c in range(nck)]

    embed = _row_blocked(_embed_body, n_pad, bn,
                         [(n_pad, df), (df, h), (1, h)], W)
    edge_mlp = _row_blocked(_edge_body, e_chk, be,
                            [(e_chk, W), (e_chk, W), (e_chk, de),
                             (h, h), (h, h), (de, h), (1, h), (h, h), (1, h)],
                            W)
    node_mlp = _row_blocked(_node_body, n_pad, bn,
                            [(n_pad, W), (n_pad, W), (n_pad, W),
                             (n_pad, W), (n_pad, W),
                             (h, h), (h, h), (1, h), (h, h), (1, h)], W)
    unembed = _row_blocked(_unembed_body, n_pad, bn,
                           [(n_pad, W), (h, h), (1, h), (h, do), (1, do)], do)

    x = embed(nf_p, emb_w.T, emb_b[None])
    for l in range(n_layers):
        ws = e1_w[l, :, :h].T
        wd = e1_w[l, :, h:2 * h].T
        we = e1_w[l, :, 2 * h:].T
        aggs = []
        for c in range(nck):
            xs, xd = gather(x, srcb[c], dstb[c])
            m = edge_mlp(xs, xd, efc[c], ws, wd, we, e1_b[l][None],
                         e2_w[l].T, e2_b[l][None])
            aggs.append(scatter(m, dstb[c], zeros_np))
        x = node_mlp(x, aggs[0][0], aggs[0][1], aggs[1][0], aggs[1][1],
                     n1_w[l, :, :h].T, n1_w[l, :, h:].T,
                     n1_b[l][None], n2_w[l].T, n2_b[l][None])
    pred = unembed(x, u1_w.T, u1_b[None], u2_w.T, u2_b[None])[:n]
    return (pred, edge_indices, edges_features)
